# baseline (speedup 1.0000x reference)
import functools

import jax
import jax.numpy as jnp
from jax import lax
from jax.experimental import pallas as pl
from jax.experimental.pallas import tpu as pltpu
from jax.experimental.pallas import tpu_sc as plsc

N = 10000
E = 320000
D = 128
L = 3

NC = 2
NS = 16
NW = NC * NS
NP = 10240
RPT = NP // NW
K = 128
NCH = 80
EPT = NCH * K
EP = NW * EPT
KS = 128
NCHS = EPT // KS
NB = 2
DW = D

_mesh = plsc.VectorSubcoreMesh(
    core_axis_name="c", subcore_axis_name="s", num_cores=NC, num_subcores=NS)

_f32 = jnp.float32
_i32 = jnp.int32


def _fill_const(ref, rows, cols, val):
  v = jnp.full((16,), val, _f32)
  nslot = cols // 16

  def body(i, carry):
    ref[i // nslot, pl.ds((i % nslot) * 16, 16)] = v
    return carry

  lax.fori_loop(0, rows * nslot, body, 0)


def _copy_idx_chunk(src_ref, off, dst_ref):
  def body(i, carry):
    dst_ref[pl.ds(i * 16, 16)] = src_ref[pl.ds(off + i * 16, 16)]
    return carry

  lax.fori_loop(0, K // 16, body, 0)


def _sc_setup_body(emb_h, tok_h, dstp_h, x_h, deg_h,
                   tok_v, rows_v, ones_v, didx_all, didx_v, deg_acc, sem):
  c = lax.axis_index("c")
  s = lax.axis_index("s")
  wid = c * NS + s

  base = wid * RPT

  def gchunk(i, carry):
    b = base + i * 64
    pltpu.sync_copy(tok_h.at[pl.ds(b, 64)], tok_v)
    pltpu.async_copy(emb_h.at[tok_v], rows_v, sem).wait()
    pltpu.sync_copy(rows_v, x_h.at[pl.ds(b, 64)])
    return carry

  lax.fori_loop(0, RPT // 64, gchunk, 0)

  _fill_const(ones_v, K, DW, 0.0)

  def zchunk(i, carry):
    pltpu.sync_copy(ones_v, deg_acc.at[pl.ds(s * RPT * NC + i * K, K)])
    return carry

  lax.fori_loop(0, (RPT * NC) // K, zchunk, 0)
  _fill_const(ones_v, K, DW, 1.0)
  pltpu.sync_copy(dstp_h.at[pl.ds(wid * EPT, EPT)], didx_all)
  plsc.subcore_barrier()

  def echunk(i, carry):
    _copy_idx_chunk(didx_all, i * K, didx_v)
    pltpu.sync_copy(ones_v, deg_acc.at[didx_v], add=True)
    return carry

  lax.fori_loop(0, NCH, echunk, 0)
  plsc.subcore_barrier()

  def ochunk(i, carry):
    r = s * RPT * NC + i * K
    pltpu.sync_copy(deg_acc.at[pl.ds(r, K)], ones_v)
    pltpu.sync_copy(ones_v, deg_h.at[c, pl.ds(r, K)])
    return carry

  lax.fori_loop(0, (RPT * NC) // K, ochunk, 0)


_SC_SETUP_SCRATCH = [
    pltpu.VMEM((64,), _i32),
    pltpu.VMEM((64, D), _f32),
    pltpu.VMEM((K, DW), _f32),
    pltpu.VMEM((EPT,), _i32),
    pltpu.VMEM((K,), _i32),
    pltpu.VMEM_SHARED((NP, DW), _f32),
    pltpu.SemaphoreType.DMA,
]

_sc_setup = pl.kernel(
    _sc_setup_body,
    out_type=[
        jax.ShapeDtypeStruct((NP, D), _f32),
        jax.ShapeDtypeStruct((NC, NP, DW), _f32),
    ],
    mesh=_mesh,
    scratch_types=_SC_SETUP_SCRATCH,
)


def _sc_segsum_body(h_h, srcp_h, dstp_h, s_h,
                    sidx_all, das, bufs, acc, gsems, ssems):
  c = lax.axis_index("c")
  s = lax.axis_index("s")
  wid = c * NS + s

  pltpu.sync_copy(srcp_h.at[pl.ds(wid * EPT, EPT)], sidx_all)

  _fill_const(bufs[0], KS, D, 0.0)

  def zchunk(i, carry):
    pltpu.sync_copy(bufs[0], acc.at[pl.ds(s * RPT * NC + i * KS, KS)])
    return carry

  lax.fori_loop(0, (RPT * NC) // KS, zchunk, 0)
  plsc.subcore_barrier()

  ebase = wid * EPT

  def didx_load(j, da):
    pltpu.sync_copy(dstp_h.at[pl.ds(ebase + j * KS, KS)], da)

  def gather(j, b):
    pltpu.async_copy(h_h.at[sidx_all.at[pl.ds(j * KS, KS)]], bufs[b], gsems[b])

  def wait_g(b):
    pltpu.make_async_copy(
        h_h.at[sidx_all.at[pl.ds(0, KS)]], bufs[b], gsems[b]).wait()

  def scat(b):
    pltpu.async_copy(bufs[b], acc.at[das[b]], ssems[b], add=True)

  def wait_s(b):
    pltpu.make_async_copy(bufs[b], acc.at[das[b]], ssems[b]).wait()

  for b in range(NB):
    didx_load(b, das[b])
    gather(b, b)

  def pipe(j4, carry):
    base = NB * j4
    for b in range(NB):
      wait_g(b)
      scat(b)
    for b in range(NB):
      wait_s(b)
      didx_load(base + NB + b, das[b])
      gather(base + NB + b, b)
    return carry

  lax.fori_loop(0, NCHS // NB - 1, pipe, 0)
  for b in range(NB):
    wait_g(b)
    scat(b)
  for b in range(NB):
    wait_s(b)
  plsc.subcore_barrier()

  def ochunk(i, carry):
    r = s * RPT * NC + i * KS
    pltpu.sync_copy(acc.at[pl.ds(r, KS)], bufs[0])
    pltpu.sync_copy(bufs[0], s_h.at[c, pl.ds(r, KS)])
    return carry

  lax.fori_loop(0, (RPT * NC) // KS, ochunk, 0)


_SC_SEGSUM_SCRATCH = [
    pltpu.VMEM((EPT,), _i32),
    [pltpu.VMEM((KS,), _i32) for _ in range(NB)],
    [pltpu.VMEM((KS, D), _f32) for _ in range(NB)],
    pltpu.VMEM_SHARED((NP, D), _f32),
    [pltpu.SemaphoreType.DMA for _ in range(NB)],
    [pltpu.SemaphoreType.DMA for _ in range(NB)],
]

_sc_segsum = pl.kernel(
    _sc_segsum_body,
    out_type=jax.ShapeDtypeStruct((NC, NP, D), _f32),
    mesh=_mesh,
    scratch_types=_SC_SEGSUM_SCRATCH,
)


_BR = 400


def _tc_h0(x, w, b, degp, bs):
  def body(x_ref, w_ref, b_ref, d_ref, bs_ref, o_ref, w1_ref):
    deg = d_ref[0, :, 0:1] + d_ref[1, :, 0:1]
    mask = (deg > 0.0).astype(_f32)
    o_ref[...] = (jnp.dot(x_ref[...], w_ref[...], preferred_element_type=_f32)
                  + b_ref[...] + mask * bs_ref[...])
    w1_ref[...] = 1.0 / jnp.maximum(deg, 1.0)

  return pl.pallas_call(
      body,
      grid=(N // _BR,),
      in_specs=[
          pl.BlockSpec((_BR, D), lambda i: (i, 0)),
          pl.BlockSpec((D, D), lambda i: (0, 0)),
          pl.BlockSpec((1, D), lambda i: (0, 0)),
          pl.BlockSpec((NC, _BR, DW), lambda i: (0, i, 0)),
          pl.BlockSpec((1, D), lambda i: (0, 0)),
      ],
      out_specs=[
          pl.BlockSpec((_BR, D), lambda i: (i, 0)),
          pl.BlockSpec((_BR, 1), lambda i: (i, 0)),
      ],
      out_shape=[
          jax.ShapeDtypeStruct((N, D), _f32),
          jax.ShapeDtypeStruct((N, 1), _f32),
      ],
  )(x, w, b, degp, bs)


def _tc_update(sab, h0p, w1, w):
  def body(s_ref, h0_ref, w1_ref, w_ref, oh_ref, or_ref):
    ssum = s_ref[0] + s_ref[1]
    h = (h0_ref[...] +
         jnp.dot(ssum, w_ref[...], preferred_element_type=_f32) * w1_ref[...])
    oh_ref[...] = h
    or_ref[...] = jnp.maximum(h, 0.0)

  rspec = pl.BlockSpec((_BR, D), lambda i: (i, 0))
  return pl.pallas_call(
      body,
      grid=(N // _BR,),
      in_specs=[
          pl.BlockSpec((NC, _BR, D), lambda i: (0, i, 0)),
          rspec,
          pl.BlockSpec((_BR, 1), lambda i: (i, 0)),
          pl.BlockSpec((D, D), lambda i: (0, 0)),
      ],
      out_specs=[rspec, rspec],
      out_shape=[
          jax.ShapeDtypeStruct((N, D), _f32),
          jax.ShapeDtypeStruct((N, D), _f32),
      ],
  )(sab, h0p, w1, w)


def _tc_pack(sab, h0p, w1, w, r0, r1):
  def body(s_ref, h0_ref, w1_ref, w_ref, r0_ref, r1_ref, onl_ref, om_ref):
    ssum = s_ref[0] + s_ref[1]
    h = (h0_ref[...] +
         jnp.dot(ssum, w_ref[...], preferred_element_type=_f32) * w1_ref[...])
    r2 = jnp.maximum(h, 0.0)
    r0v = r0_ref[...]
    r1v = r1_ref[...]
    onl_ref[0] = r0v
    onl_ref[1] = r1v
    onl_ref[2] = r2
    om_ref[...] = jnp.maximum(jnp.maximum(r0v, r1v), r2)

  rspec = pl.BlockSpec((_BR, D), lambda i: (i, 0))
  return pl.pallas_call(
      body,
      grid=(N // _BR,),
      in_specs=[
          pl.BlockSpec((NC, _BR, D), lambda i: (0, i, 0)),
          rspec,
          pl.BlockSpec((_BR, 1), lambda i: (i, 0)),
          pl.BlockSpec((D, D), lambda i: (0, 0)),
          rspec,
          rspec,
      ],
      out_specs=[
          pl.BlockSpec((L, _BR, D), lambda i: (0, i, 0)),
          rspec,
      ],
      out_shape=[
          jax.ShapeDtypeStruct((L, N, D), _f32),
          jax.ShapeDtypeStruct((N, D), _f32),
      ],
  )(sab, h0p, w1, w, r0, r1)


def kernel(node_tokens, edge_index, emb, W_c, b_c, W_sum, b_sum):
  tok = node_tokens.astype(_i32)
  src = edge_index[0].astype(_i32)
  dst = edge_index[1].astype(_i32)
  tokp = jnp.concatenate([tok, jnp.zeros((NP - N,), _i32)])
  ppt = EPT - E // NW
  pad_dst = jnp.tile(N + jnp.arange(ppt, dtype=_i32) % (NP - N), (NW, 1))
  pad_src = jnp.tile(jnp.arange(ppt, dtype=_i32), (NW, 1))
  srcp = jnp.concatenate([src.reshape(NW, E // NW), pad_src], axis=1).reshape(-1)
  dstp = jnp.concatenate([dst.reshape(NW, E // NW), pad_dst], axis=1).reshape(-1)
  b_c2 = b_c.reshape(1, D).astype(_f32)
  b_s2 = b_sum.reshape(1, D).astype(_f32)

  x, degp = _sc_setup(emb, tokp, dstp)
  h0p, w1 = _tc_h0(x, W_c, b_c2, degp, b_s2)
  h = h0p
  rs = []
  for _ in range(L - 1):
    sab = _sc_segsum(h, srcp, dstp)
    h, r = _tc_update(sab, h0p, w1, W_sum)
    rs.append(r)
  sab = _sc_segsum(h, srcp, dstp)
  nl, m = _tc_pack(sab, h0p, w1, W_sum, rs[0], rs[1])
  return nl, m


# speedup vs baseline: 1.1332x; 1.1332x over previous
import functools

import jax
import jax.numpy as jnp
from jax import lax
from jax.experimental import pallas as pl
from jax.experimental.pallas import tpu as pltpu
from jax.experimental.pallas import tpu_sc as plsc

N = 10000
E = 320000
D = 128
L = 3

NC = 2
NS = 16
NW = NC * NS
NP = 10240
RPT = NP // NW
K = 128
NCH = 80
EPT = NCH * K
EP = NW * EPT
KS = 128
NCHS = EPT // KS
NB = 2
DW = D

_mesh = plsc.VectorSubcoreMesh(
    core_axis_name="c", subcore_axis_name="s", num_cores=NC, num_subcores=NS)

_f32 = jnp.float32
_i32 = jnp.int32


def _fill_const(ref, rows, cols, val):
  v = jnp.full((16,), val, _f32)
  nslot = cols // 16

  def body(i, carry):
    ref[i // nslot, pl.ds((i % nslot) * 16, 16)] = v
    return carry

  lax.fori_loop(0, rows * nslot, body, 0)


def _copy_idx_chunk(src_ref, off, dst_ref):
  def body(i, carry):
    dst_ref[pl.ds(i * 16, 16)] = src_ref[pl.ds(off + i * 16, 16)]
    return carry

  lax.fori_loop(0, K // 16, body, 0)


def _sc_setup_body(emb_h, tok_h, dstp_h, x_h, deg_h,
                   tok_v, rows_v, ones_v, didx_all, didx_v, deg_acc, sem):
  c = lax.axis_index("c")
  s = lax.axis_index("s")
  wid = c * NS + s

  base = wid * RPT

  def gchunk(i, carry):
    b = base + i * 64
    pltpu.sync_copy(tok_h.at[pl.ds(b, 64)], tok_v)
    pltpu.async_copy(emb_h.at[tok_v], rows_v, sem).wait()
    pltpu.sync_copy(rows_v, x_h.at[pl.ds(b, 64)])
    return carry

  lax.fori_loop(0, RPT // 64, gchunk, 0)

  _fill_const(ones_v, K, DW, 0.0)

  def zchunk(i, carry):
    pltpu.sync_copy(ones_v, deg_acc.at[pl.ds(s * RPT * NC + i * K, K)])
    return carry

  lax.fori_loop(0, (RPT * NC) // K, zchunk, 0)
  _fill_const(ones_v, K, DW, 1.0)
  pltpu.sync_copy(dstp_h.at[pl.ds(wid * EPT, EPT)], didx_all)
  plsc.subcore_barrier()

  def echunk(i, carry):
    _copy_idx_chunk(didx_all, i * K, didx_v)
    pltpu.sync_copy(ones_v, deg_acc.at[didx_v], add=True)
    return carry

  lax.fori_loop(0, NCH, echunk, 0)
  plsc.subcore_barrier()

  def ochunk(i, carry):
    r = s * RPT * NC + i * K
    pltpu.sync_copy(deg_acc.at[pl.ds(r, K)], ones_v)
    pltpu.sync_copy(ones_v, deg_h.at[c, pl.ds(r, K)])
    return carry

  lax.fori_loop(0, (RPT * NC) // K, ochunk, 0)


_SC_SETUP_SCRATCH = [
    pltpu.VMEM((64,), _i32),
    pltpu.VMEM((64, D), _f32),
    pltpu.VMEM((K, DW), _f32),
    pltpu.VMEM((EPT,), _i32),
    pltpu.VMEM((K,), _i32),
    pltpu.VMEM_SHARED((NP, DW), _f32),
    pltpu.SemaphoreType.DMA,
]

_sc_setup = pl.kernel(
    _sc_setup_body,
    out_type=[
        jax.ShapeDtypeStruct((NP, D), _f32),
        jax.ShapeDtypeStruct((NC, NP, DW), _f32),
    ],
    mesh=_mesh,
    scratch_types=_SC_SETUP_SCRATCH,
)


def _sc_segsum_body(h_h, srcp_h, dstp_h, s_h,
                    sidx_all, da0, da1, buf_a, buf_b,
                    acc, sem_a, sem_b):
  c = lax.axis_index("c")
  s = lax.axis_index("s")
  wid = c * NS + s

  pltpu.sync_copy(srcp_h.at[pl.ds(wid * EPT, EPT)], sidx_all)

  _fill_const(buf_a, K, D, 0.0)

  def zchunk(i, carry):
    pltpu.sync_copy(buf_a, acc.at[pl.ds(s * RPT * NC + i * K, K)])
    return carry

  lax.fori_loop(0, (RPT * NC) // K, zchunk, 0)
  plsc.subcore_barrier()

  ebase = wid * EPT

  def didx_load(j, da):
    pltpu.sync_copy(dstp_h.at[pl.ds(ebase + j * K, K)], da)

  def gather(j, buf, sem):
    return pltpu.async_copy(h_h.at[sidx_all.at[pl.ds(j * K, K)]], buf, sem)

  def wait(buf, sem):
    pltpu.make_async_copy(h_h.at[sidx_all.at[pl.ds(0, K)]], buf, sem).wait()

  def scat(buf, didx):
    pltpu.sync_copy(buf, acc.at[didx], add=True)

  didx_load(0, da0)
  gather(0, buf_a, sem_a)

  def pipe(jj, carry):
    j = 2 * jj
    gather(j + 1, buf_b, sem_b)
    didx_load(j + 1, da1)
    wait(buf_a, sem_a)
    scat(buf_a, da0)
    gather(j + 2, buf_a, sem_a)
    didx_load(j + 2, da0)
    wait(buf_b, sem_b)
    scat(buf_b, da1)
    return carry

  lax.fori_loop(0, (NCH - 2) // 2, pipe, 0)
  gather(NCH - 1, buf_b, sem_b)
  didx_load(NCH - 1, da1)
  wait(buf_a, sem_a)
  scat(buf_a, da0)
  wait(buf_b, sem_b)
  scat(buf_b, da1)
  plsc.subcore_barrier()

  def ochunk(i, carry):
    r = s * RPT * NC + i * K
    pltpu.sync_copy(acc.at[pl.ds(r, K)], buf_a)
    pltpu.sync_copy(buf_a, s_h.at[c, pl.ds(r, K)])
    return carry

  lax.fori_loop(0, (RPT * NC) // K, ochunk, 0)


_SC_SEGSUM_SCRATCH = [
    pltpu.VMEM((EPT,), _i32),
    pltpu.VMEM((K,), _i32),
    pltpu.VMEM((K,), _i32),
    pltpu.VMEM((K, D), _f32),
    pltpu.VMEM((K, D), _f32),
    pltpu.VMEM_SHARED((NP, D), _f32),
    pltpu.SemaphoreType.DMA,
    pltpu.SemaphoreType.DMA,
]

_sc_segsum = pl.kernel(
    _sc_segsum_body,
    out_type=jax.ShapeDtypeStruct((NC, NP, D), _f32),
    mesh=_mesh,
    scratch_types=_SC_SEGSUM_SCRATCH,
)


_BR = 400


def _tc_h0(x, w, b, degp, bs):
  def body(x_ref, w_ref, b_ref, d_ref, bs_ref, o_ref, w1_ref):
    deg = d_ref[0, :, 0:1] + d_ref[1, :, 0:1]
    mask = (deg > 0.0).astype(_f32)
    o_ref[...] = (jnp.dot(x_ref[...], w_ref[...], preferred_element_type=_f32)
                  + b_ref[...] + mask * bs_ref[...])
    w1_ref[...] = 1.0 / jnp.maximum(deg, 1.0)

  return pl.pallas_call(
      body,
      grid=(N // _BR,),
      in_specs=[
          pl.BlockSpec((_BR, D), lambda i: (i, 0)),
          pl.BlockSpec((D, D), lambda i: (0, 0)),
          pl.BlockSpec((1, D), lambda i: (0, 0)),
          pl.BlockSpec((NC, _BR, DW), lambda i: (0, i, 0)),
          pl.BlockSpec((1, D), lambda i: (0, 0)),
      ],
      out_specs=[
          pl.BlockSpec((_BR, D), lambda i: (i, 0)),
          pl.BlockSpec((_BR, 1), lambda i: (i, 0)),
      ],
      out_shape=[
          jax.ShapeDtypeStruct((N, D), _f32),
          jax.ShapeDtypeStruct((N, 1), _f32),
      ],
  )(x, w, b, degp, bs)


def _tc_update(sab, h0p, w1, w):
  def body(s_ref, h0_ref, w1_ref, w_ref, oh_ref, or_ref):
    ssum = s_ref[0] + s_ref[1]
    h = (h0_ref[...] +
         jnp.dot(ssum, w_ref[...], preferred_element_type=_f32) * w1_ref[...])
    oh_ref[...] = h
    or_ref[...] = jnp.maximum(h, 0.0)

  rspec = pl.BlockSpec((_BR, D), lambda i: (i, 0))
  return pl.pallas_call(
      body,
      grid=(N // _BR,),
      in_specs=[
          pl.BlockSpec((NC, _BR, D), lambda i: (0, i, 0)),
          rspec,
          pl.BlockSpec((_BR, 1), lambda i: (i, 0)),
          pl.BlockSpec((D, D), lambda i: (0, 0)),
      ],
      out_specs=[rspec, rspec],
      out_shape=[
          jax.ShapeDtypeStruct((N, D), _f32),
          jax.ShapeDtypeStruct((N, D), _f32),
      ],
  )(sab, h0p, w1, w)


def _tc_pack(sab, h0p, w1, w, r0, r1):
  def body(s_ref, h0_ref, w1_ref, w_ref, r0_ref, r1_ref, onl_ref, om_ref):
    ssum = s_ref[0] + s_ref[1]
    h = (h0_ref[...] +
         jnp.dot(ssum, w_ref[...], preferred_element_type=_f32) * w1_ref[...])
    r2 = jnp.maximum(h, 0.0)
    r0v = r0_ref[...]
    r1v = r1_ref[...]
    onl_ref[0] = r0v
    onl_ref[1] = r1v
    onl_ref[2] = r2
    om_ref[...] = jnp.maximum(jnp.maximum(r0v, r1v), r2)

  rspec = pl.BlockSpec((_BR, D), lambda i: (i, 0))
  return pl.pallas_call(
      body,
      grid=(N // _BR,),
      in_specs=[
          pl.BlockSpec((NC, _BR, D), lambda i: (0, i, 0)),
          rspec,
          pl.BlockSpec((_BR, 1), lambda i: (i, 0)),
          pl.BlockSpec((D, D), lambda i: (0, 0)),
          rspec,
          rspec,
      ],
      out_specs=[
          pl.BlockSpec((L, _BR, D), lambda i: (0, i, 0)),
          rspec,
      ],
      out_shape=[
          jax.ShapeDtypeStruct((L, N, D), _f32),
          jax.ShapeDtypeStruct((N, D), _f32),
      ],
  )(sab, h0p, w1, w, r0, r1)


def kernel(node_tokens, edge_index, emb, W_c, b_c, W_sum, b_sum):
  tok = node_tokens.astype(_i32)
  src = edge_index[0].astype(_i32)
  dst = edge_index[1].astype(_i32)
  tokp = jnp.concatenate([tok, jnp.zeros((NP - N,), _i32)])
  ppt = EPT - E // NW
  pad_dst = jnp.tile(N + jnp.arange(ppt, dtype=_i32) % (NP - N), (NW, 1))
  pad_src = jnp.tile(jnp.arange(ppt, dtype=_i32), (NW, 1))
  srcp = jnp.concatenate([src.reshape(NW, E // NW), pad_src], axis=1).reshape(-1)
  dstp = jnp.concatenate([dst.reshape(NW, E // NW), pad_dst], axis=1).reshape(-1)
  b_c2 = b_c.reshape(1, D).astype(_f32)
  b_s2 = b_sum.reshape(1, D).astype(_f32)

  x, degp = _sc_setup(emb, tokp, dstp)
  h0p, w1 = _tc_h0(x, W_c, b_c2, degp, b_s2)
  h = h0p
  rs = []
  for _ in range(L - 1):
    sab = _sc_segsum(h, srcp, dstp)
    h, r = _tc_update(sab, h0p, w1, W_sum)
    rs.append(r)
  sab = _sc_segsum(h, srcp, dstp)
  nl, m = _tc_pack(sab, h0p, w1, W_sum, rs[0], rs[1])
  return nl, m


# speedup vs baseline: 1.2012x; 1.0600x over previous
import functools

import jax
import jax.numpy as jnp
from jax import lax
from jax.experimental import pallas as pl
from jax.experimental.pallas import tpu as pltpu
from jax.experimental.pallas import tpu_sc as plsc

N = 10000
E = 320000
D = 128
L = 3

NC = 2
NS = 16
NW = NC * NS
NP = 10240
RPT = NP // NW
K = 128
NCH = 80
EPT = NCH * K
EP = NW * EPT
KS = 128
NCHS = EPT // KS
NB = 2
DW = D

_mesh = plsc.VectorSubcoreMesh(
    core_axis_name="c", subcore_axis_name="s", num_cores=NC, num_subcores=NS)

_f32 = jnp.float32
_i32 = jnp.int32


def _fill_const(ref, rows, cols, val):
  v = jnp.full((16,), val, _f32)
  nslot = cols // 16

  def body(i, carry):
    ref[i // nslot, pl.ds((i % nslot) * 16, 16)] = v
    return carry

  lax.fori_loop(0, rows * nslot, body, 0)


def _copy_idx_chunk(src_ref, off, dst_ref):
  def body(i, carry):
    dst_ref[pl.ds(i * 16, 16)] = src_ref[pl.ds(off + i * 16, 16)]
    return carry

  lax.fori_loop(0, K // 16, body, 0)


def _sc_setup_body(emb_h, tok_h, dstp_h, x_h, deg_h,
                   tok_v, rows_v, ones_v, didx_all, didx_v, deg_acc, sem):
  c = lax.axis_index("c")
  s = lax.axis_index("s")
  wid = c * NS + s

  base = wid * RPT

  def gchunk(i, carry):
    b = base + i * 64
    pltpu.sync_copy(tok_h.at[pl.ds(b, 64)], tok_v)
    pltpu.async_copy(emb_h.at[tok_v], rows_v, sem).wait()
    pltpu.sync_copy(rows_v, x_h.at[pl.ds(b, 64)])
    return carry

  lax.fori_loop(0, RPT // 64, gchunk, 0)

  _fill_const(ones_v, K, DW, 0.0)

  def zchunk(i, carry):
    pltpu.sync_copy(ones_v, deg_acc.at[pl.ds(s * RPT * NC + i * K, K)])
    return carry

  lax.fori_loop(0, (RPT * NC) // K, zchunk, 0)
  _fill_const(ones_v, K, DW, 1.0)
  pltpu.sync_copy(dstp_h.at[pl.ds(wid * EPT, EPT)], didx_all)
  plsc.subcore_barrier()

  def echunk(i, carry):
    _copy_idx_chunk(didx_all, i * K, didx_v)
    pltpu.sync_copy(ones_v, deg_acc.at[didx_v], add=True)
    return carry

  lax.fori_loop(0, NCH, echunk, 0)
  plsc.subcore_barrier()

  def ochunk(i, carry):
    r = s * RPT * NC + i * K
    pltpu.sync_copy(deg_acc.at[pl.ds(r, K)], ones_v)
    pltpu.sync_copy(ones_v, deg_h.at[c, pl.ds(r, K)])
    return carry

  lax.fori_loop(0, (RPT * NC) // K, ochunk, 0)


_SC_SETUP_SCRATCH = [
    pltpu.VMEM((64,), _i32),
    pltpu.VMEM((64, D), _f32),
    pltpu.VMEM((K, DW), _f32),
    pltpu.VMEM((EPT,), _i32),
    pltpu.VMEM((K,), _i32),
    pltpu.VMEM_SHARED((NP, DW), _f32),
    pltpu.SemaphoreType.DMA,
]

_sc_setup = pl.kernel(
    _sc_setup_body,
    out_type=[
        jax.ShapeDtypeStruct((NP, D), _f32),
        jax.ShapeDtypeStruct((NC, NP, DW), _f32),
    ],
    mesh=_mesh,
    scratch_types=_SC_SETUP_SCRATCH,
)


def _sc_segsum_body(h_h, srcp_h, dstp_h, s_h,
                    sidx_all, da0, da1, buf_a, buf_b,
                    acc, sem_a, sem_b):
  c = lax.axis_index("c")
  s = lax.axis_index("s")
  wid = c * NS + s

  pltpu.sync_copy(srcp_h.at[pl.ds(wid * EPT, EPT)], sidx_all)

  _fill_const(buf_a, K, D, 0.0)

  def zchunk(i, carry):
    pltpu.sync_copy(buf_a, acc.at[pl.ds(s * RPT * NC + i * K, K)])
    return carry

  lax.fori_loop(0, (RPT * NC) // K, zchunk, 0)
  plsc.subcore_barrier()

  ebase = wid * EPT

  def didx_load(j, da):
    pltpu.sync_copy(dstp_h.at[pl.ds(ebase + j * K, K)], da)

  def gather(j, buf, sem):
    return pltpu.async_copy(h_h.at[sidx_all.at[pl.ds(j * K, K)]], buf, sem)

  def wait(buf, sem):
    pltpu.make_async_copy(h_h.at[sidx_all.at[pl.ds(0, K)]], buf, sem).wait()

  def scat(buf, didx):
    pltpu.sync_copy(buf, acc.at[didx], add=True)

  didx_load(0, da0)
  gather(0, buf_a, sem_a)

  def pipe(jj, carry):
    j = 2 * jj
    gather(j + 1, buf_b, sem_b)
    didx_load(j + 1, da1)
    wait(buf_a, sem_a)
    scat(buf_a, da0)
    gather(j + 2, buf_a, sem_a)
    didx_load(j + 2, da0)
    wait(buf_b, sem_b)
    scat(buf_b, da1)
    return carry

  lax.fori_loop(0, (NCH - 2) // 2, pipe, 0)
  gather(NCH - 1, buf_b, sem_b)
  didx_load(NCH - 1, da1)
  wait(buf_a, sem_a)
  scat(buf_a, da0)
  wait(buf_b, sem_b)
  scat(buf_b, da1)
  plsc.subcore_barrier()

  def ochunk(i, carry):
    r = s * RPT * NC + i * K
    pltpu.sync_copy(acc.at[pl.ds(r, K)], buf_a)
    pltpu.sync_copy(buf_a, s_h.at[c, pl.ds(r, K)])
    return carry

  lax.fori_loop(0, (RPT * NC) // K, ochunk, 0)


_SC_SEGSUM_SCRATCH = [
    pltpu.VMEM((EPT,), _i32),
    pltpu.VMEM((K,), _i32),
    pltpu.VMEM((K,), _i32),
    pltpu.VMEM((K, D), _f32),
    pltpu.VMEM((K, D), _f32),
    pltpu.VMEM_SHARED((NP, D), _f32),
    pltpu.SemaphoreType.DMA,
    pltpu.SemaphoreType.DMA,
]

_sc_segsum = pl.kernel(
    _sc_segsum_body,
    out_type=jax.ShapeDtypeStruct((NC, NP, D), _f32),
    mesh=_mesh,
    scratch_types=_SC_SEGSUM_SCRATCH,
)


_BR = 1000


def _tc_h0(x, w, b, degp, bs):
  def body(x_ref, w_ref, b_ref, d_ref, bs_ref, o_ref, w1_ref):
    deg = d_ref[0, :, 0:1] + d_ref[1, :, 0:1]
    mask = (deg > 0.0).astype(_f32)
    o_ref[...] = (jnp.dot(x_ref[...], w_ref[...], preferred_element_type=_f32)
                  + b_ref[...] + mask * bs_ref[...])
    w1_ref[...] = 1.0 / jnp.maximum(deg, 1.0)

  return pl.pallas_call(
      body,
      grid=(N // _BR,),
      in_specs=[
          pl.BlockSpec((_BR, D), lambda i: (i, 0)),
          pl.BlockSpec((D, D), lambda i: (0, 0)),
          pl.BlockSpec((1, D), lambda i: (0, 0)),
          pl.BlockSpec((NC, _BR, DW), lambda i: (0, i, 0)),
          pl.BlockSpec((1, D), lambda i: (0, 0)),
      ],
      out_specs=[
          pl.BlockSpec((_BR, D), lambda i: (i, 0)),
          pl.BlockSpec((_BR, 1), lambda i: (i, 0)),
      ],
      out_shape=[
          jax.ShapeDtypeStruct((N, D), _f32),
          jax.ShapeDtypeStruct((N, 1), _f32),
      ],
  )(x, w, b, degp, bs)


def _tc_update(sab, h0p, w1, w):
  def body(s_ref, h0_ref, w1_ref, w_ref, oh_ref, or_ref):
    ssum = s_ref[0] + s_ref[1]
    h = (h0_ref[...] +
         jnp.dot(ssum, w_ref[...], preferred_element_type=_f32) * w1_ref[...])
    oh_ref[...] = h
    or_ref[...] = jnp.maximum(h, 0.0)

  rspec = pl.BlockSpec((_BR, D), lambda i: (i, 0))
  return pl.pallas_call(
      body,
      grid=(N // _BR,),
      in_specs=[
          pl.BlockSpec((NC, _BR, D), lambda i: (0, i, 0)),
          rspec,
          pl.BlockSpec((_BR, 1), lambda i: (i, 0)),
          pl.BlockSpec((D, D), lambda i: (0, 0)),
      ],
      out_specs=[rspec, rspec],
      out_shape=[
          jax.ShapeDtypeStruct((N, D), _f32),
          jax.ShapeDtypeStruct((N, D), _f32),
      ],
  )(sab, h0p, w1, w)


def _tc_pack(sab, h0p, w1, w, r0, r1):
  def body(s_ref, h0_ref, w1_ref, w_ref, r0_ref, r1_ref, onl_ref, om_ref):
    ssum = s_ref[0] + s_ref[1]
    h = (h0_ref[...] +
         jnp.dot(ssum, w_ref[...], preferred_element_type=_f32) * w1_ref[...])
    r2 = jnp.maximum(h, 0.0)
    r0v = r0_ref[...]
    r1v = r1_ref[...]
    onl_ref[0] = r0v
    onl_ref[1] = r1v
    onl_ref[2] = r2
    om_ref[...] = jnp.maximum(jnp.maximum(r0v, r1v), r2)

  rspec = pl.BlockSpec((_BR, D), lambda i: (i, 0))
  return pl.pallas_call(
      body,
      grid=(N // _BR,),
      in_specs=[
          pl.BlockSpec((NC, _BR, D), lambda i: (0, i, 0)),
          rspec,
          pl.BlockSpec((_BR, 1), lambda i: (i, 0)),
          pl.BlockSpec((D, D), lambda i: (0, 0)),
          rspec,
          rspec,
      ],
      out_specs=[
          pl.BlockSpec((L, _BR, D), lambda i: (0, i, 0)),
          rspec,
      ],
      out_shape=[
          jax.ShapeDtypeStruct((L, N, D), _f32),
          jax.ShapeDtypeStruct((N, D), _f32),
      ],
  )(sab, h0p, w1, w, r0, r1)


def kernel(node_tokens, edge_index, emb, W_c, b_c, W_sum, b_sum):
  tok = node_tokens.astype(_i32)
  src = edge_index[0].astype(_i32)
  dst = edge_index[1].astype(_i32)
  tokp = jnp.concatenate([tok, jnp.zeros((NP - N,), _i32)])
  ppt = EPT - E // NW
  pad_dst = jnp.tile(N + jnp.arange(ppt, dtype=_i32) % (NP - N), (NW, 1))
  pad_src = jnp.tile(jnp.arange(ppt, dtype=_i32), (NW, 1))
  srcp = jnp.concatenate([src.reshape(NW, E // NW), pad_src], axis=1).reshape(-1)
  dstp = jnp.concatenate([dst.reshape(NW, E // NW), pad_dst], axis=1).reshape(-1)
  b_c2 = b_c.reshape(1, D).astype(_f32)
  b_s2 = b_sum.reshape(1, D).astype(_f32)

  x, degp = _sc_setup(emb, tokp, dstp)
  h0p, w1 = _tc_h0(x, W_c, b_c2, degp, b_s2)
  h = h0p
  rs = []
  for _ in range(L - 1):
    sab = _sc_segsum(h, srcp, dstp)
    h, r = _tc_update(sab, h0p, w1, W_sum)
    rs.append(r)
  sab = _sc_segsum(h, srcp, dstp)
  nl, m = _tc_pack(sab, h0p, w1, W_sum, rs[0], rs[1])
  return nl, m


# speedup vs baseline: 1.2160x; 1.0124x over previous
import functools

import jax
import jax.numpy as jnp
from jax import lax
from jax.experimental import pallas as pl
from jax.experimental.pallas import tpu as pltpu
from jax.experimental.pallas import tpu_sc as plsc

N = 10000
E = 320000
D = 128
L = 3

NC = 2
NS = 16
NW = NC * NS
NP = 10240
RPT = NP // NW
K = 128
NCH = 80
EPT = NCH * K
EP = NW * EPT
KS = 128
NCHS = EPT // KS
NB = 2
DW = D

_mesh = plsc.VectorSubcoreMesh(
    core_axis_name="c", subcore_axis_name="s", num_cores=NC, num_subcores=NS)

_f32 = jnp.float32
_i32 = jnp.int32


def _fill_const(ref, rows, cols, val):
  v = jnp.full((16,), val, _f32)
  nslot = cols // 16

  def body(i, carry):
    ref[i // nslot, pl.ds((i % nslot) * 16, 16)] = v
    return carry

  lax.fori_loop(0, rows * nslot, body, 0)


def _copy_idx_chunk(src_ref, off, dst_ref):
  def body(i, carry):
    dst_ref[pl.ds(i * 16, 16)] = src_ref[pl.ds(off + i * 16, 16)]
    return carry

  lax.fori_loop(0, K // 16, body, 0)


def _sc_setup_body(emb_h, tok_h, dstp_h, x_h, deg_h,
                   tok_v, rows_v, ones_v, didx_all, didx_v, deg_acc, sem):
  c = lax.axis_index("c")
  s = lax.axis_index("s")
  wid = c * NS + s

  base = wid * RPT

  def gchunk(i, carry):
    b = base + i * 64
    pltpu.sync_copy(tok_h.at[pl.ds(b, 64)], tok_v)
    pltpu.async_copy(emb_h.at[tok_v], rows_v, sem).wait()
    pltpu.sync_copy(rows_v, x_h.at[pl.ds(b, 64)])
    return carry

  lax.fori_loop(0, RPT // 64, gchunk, 0)

  _fill_const(ones_v, K, DW, 0.0)

  def zchunk(i, carry):
    pltpu.sync_copy(ones_v, deg_acc.at[pl.ds(s * RPT * NC + i * K, K)])
    return carry

  lax.fori_loop(0, (RPT * NC) // K, zchunk, 0)
  _fill_const(ones_v, K, DW, 1.0)
  pltpu.sync_copy(dstp_h.at[pl.ds(wid * EPT, EPT)], didx_all)
  plsc.subcore_barrier()

  def echunk(i, carry):
    _copy_idx_chunk(didx_all, i * K, didx_v)
    pltpu.sync_copy(ones_v, deg_acc.at[didx_v], add=True)
    return carry

  lax.fori_loop(0, NCH, echunk, 0)
  plsc.subcore_barrier()

  def ochunk(i, carry):
    r = s * RPT * NC + i * K
    pltpu.sync_copy(deg_acc.at[pl.ds(r, K)], ones_v)
    pltpu.sync_copy(ones_v, deg_h.at[c, pl.ds(r, K)])
    return carry

  lax.fori_loop(0, (RPT * NC) // K, ochunk, 0)


_SC_SETUP_SCRATCH = [
    pltpu.VMEM((64,), _i32),
    pltpu.VMEM((64, D), _f32),
    pltpu.VMEM((K, DW), _f32),
    pltpu.VMEM((EPT,), _i32),
    pltpu.VMEM((K,), _i32),
    pltpu.VMEM_SHARED((NP, DW), _f32),
    pltpu.SemaphoreType.DMA,
]

_sc_setup = pl.kernel(
    _sc_setup_body,
    out_type=[
        jax.ShapeDtypeStruct((NP, D), _f32),
        jax.ShapeDtypeStruct((NC, NP, DW), _f32),
    ],
    mesh=_mesh,
    scratch_types=_SC_SETUP_SCRATCH,
)


def _sc_segsum_body(h_h, srcp_h, dstp_h, s_h,
                    sidx_all, da0, da1, buf_a, buf_b,
                    acc, sem_a, sem_b):
  c = lax.axis_index("c")
  s = lax.axis_index("s")
  wid = c * NS + s

  pltpu.sync_copy(srcp_h.at[pl.ds(wid * EPT, EPT)], sidx_all)

  _fill_const(buf_a, K, D, 0.0)

  def zchunk(i, carry):
    pltpu.sync_copy(buf_a, acc.at[pl.ds(s * RPT * NC + i * K, K)])
    return carry

  lax.fori_loop(0, (RPT * NC) // K, zchunk, 0)
  plsc.subcore_barrier()

  ebase = wid * EPT

  def didx_load(j, da):
    pltpu.sync_copy(dstp_h.at[pl.ds(ebase + j * K, K)], da)

  def gather(j, buf, sem):
    return pltpu.async_copy(h_h.at[sidx_all.at[pl.ds(j * K, K)]], buf, sem)

  def wait(buf, sem):
    pltpu.make_async_copy(h_h.at[sidx_all.at[pl.ds(0, K)]], buf, sem).wait()

  def scat(buf, didx):
    pltpu.sync_copy(buf, acc.at[didx], add=True)

  didx_load(0, da0)
  gather(0, buf_a, sem_a)

  def pipe(jj, carry):
    j = 2 * jj
    gather(j + 1, buf_b, sem_b)
    didx_load(j + 1, da1)
    wait(buf_a, sem_a)
    scat(buf_a, da0)
    gather(j + 2, buf_a, sem_a)
    didx_load(j + 2, da0)
    wait(buf_b, sem_b)
    scat(buf_b, da1)
    return carry

  lax.fori_loop(0, (NCH - 2) // 2, pipe, 0)
  gather(NCH - 1, buf_b, sem_b)
  didx_load(NCH - 1, da1)
  wait(buf_a, sem_a)
  scat(buf_a, da0)
  wait(buf_b, sem_b)
  scat(buf_b, da1)
  plsc.subcore_barrier()

  def ochunk(i, carry):
    r = s * RPT * NC + i * K
    pltpu.sync_copy(acc.at[pl.ds(r, K)], buf_a)
    pltpu.sync_copy(buf_a, s_h.at[c, pl.ds(r, K)])
    return carry

  lax.fori_loop(0, (RPT * NC) // K, ochunk, 0)


_SC_SEGSUM_SCRATCH = [
    pltpu.VMEM((EPT,), _i32),
    pltpu.VMEM((K,), _i32),
    pltpu.VMEM((K,), _i32),
    pltpu.VMEM((K, D), _f32),
    pltpu.VMEM((K, D), _f32),
    pltpu.VMEM_SHARED((NP, D), _f32),
    pltpu.SemaphoreType.DMA,
    pltpu.SemaphoreType.DMA,
]

_sc_segsum = pl.kernel(
    _sc_segsum_body,
    out_type=jax.ShapeDtypeStruct((NC, NP, D), _f32),
    mesh=_mesh,
    scratch_types=_SC_SEGSUM_SCRATCH,
)


_BR = 2000


def _tc_h0(x, w, b, degp, bs):
  def body(x_ref, w_ref, b_ref, d_ref, bs_ref, o_ref, w1_ref):
    deg = d_ref[0, :, 0:1] + d_ref[1, :, 0:1]
    mask = (deg > 0.0).astype(_f32)
    o_ref[...] = (jnp.dot(x_ref[...], w_ref[...], preferred_element_type=_f32)
                  + b_ref[...] + mask * bs_ref[...])
    w1_ref[...] = 1.0 / jnp.maximum(deg, 1.0)

  return pl.pallas_call(
      body,
      grid=(N // _BR,),
      in_specs=[
          pl.BlockSpec((_BR, D), lambda i: (i, 0)),
          pl.BlockSpec((D, D), lambda i: (0, 0)),
          pl.BlockSpec((1, D), lambda i: (0, 0)),
          pl.BlockSpec((NC, _BR, DW), lambda i: (0, i, 0)),
          pl.BlockSpec((1, D), lambda i: (0, 0)),
      ],
      out_specs=[
          pl.BlockSpec((_BR, D), lambda i: (i, 0)),
          pl.BlockSpec((_BR, 1), lambda i: (i, 0)),
      ],
      out_shape=[
          jax.ShapeDtypeStruct((N, D), _f32),
          jax.ShapeDtypeStruct((N, 1), _f32),
      ],
  )(x, w, b, degp, bs)


def _tc_update(sab, h0p, w1, w):
  def body(s_ref, h0_ref, w1_ref, w_ref, oh_ref, or_ref):
    ssum = s_ref[0] + s_ref[1]
    h = (h0_ref[...] +
         jnp.dot(ssum, w_ref[...], preferred_element_type=_f32) * w1_ref[...])
    oh_ref[...] = h
    or_ref[...] = jnp.maximum(h, 0.0)

  rspec = pl.BlockSpec((_BR, D), lambda i: (i, 0))
  return pl.pallas_call(
      body,
      grid=(N // _BR,),
      in_specs=[
          pl.BlockSpec((NC, _BR, D), lambda i: (0, i, 0)),
          rspec,
          pl.BlockSpec((_BR, 1), lambda i: (i, 0)),
          pl.BlockSpec((D, D), lambda i: (0, 0)),
      ],
      out_specs=[rspec, rspec],
      out_shape=[
          jax.ShapeDtypeStruct((N, D), _f32),
          jax.ShapeDtypeStruct((N, D), _f32),
      ],
  )(sab, h0p, w1, w)


def _tc_pack(sab, h0p, w1, w, r0, r1):
  def body(s_ref, h0_ref, w1_ref, w_ref, r0_ref, r1_ref, onl_ref, om_ref):
    ssum = s_ref[0] + s_ref[1]
    h = (h0_ref[...] +
         jnp.dot(ssum, w_ref[...], preferred_element_type=_f32) * w1_ref[...])
    r2 = jnp.maximum(h, 0.0)
    r0v = r0_ref[...]
    r1v = r1_ref[...]
    onl_ref[0] = r0v
    onl_ref[1] = r1v
    onl_ref[2] = r2
    om_ref[...] = jnp.maximum(jnp.maximum(r0v, r1v), r2)

  rspec = pl.BlockSpec((_BR, D), lambda i: (i, 0))
  return pl.pallas_call(
      body,
      grid=(N // _BR,),
      in_specs=[
          pl.BlockSpec((NC, _BR, D), lambda i: (0, i, 0)),
          rspec,
          pl.BlockSpec((_BR, 1), lambda i: (i, 0)),
          pl.BlockSpec((D, D), lambda i: (0, 0)),
          rspec,
          rspec,
      ],
      out_specs=[
          pl.BlockSpec((L, _BR, D), lambda i: (0, i, 0)),
          rspec,
      ],
      out_shape=[
          jax.ShapeDtypeStruct((L, N, D), _f32),
          jax.ShapeDtypeStruct((N, D), _f32),
      ],
  )(sab, h0p, w1, w, r0, r1)


def kernel(node_tokens, edge_index, emb, W_c, b_c, W_sum, b_sum):
  tok = node_tokens.astype(_i32)
  src = edge_index[0].astype(_i32)
  dst = edge_index[1].astype(_i32)
  tokp = jnp.concatenate([tok, jnp.zeros((NP - N,), _i32)])
  ppt = EPT - E // NW
  pad_dst = jnp.tile(N + jnp.arange(ppt, dtype=_i32) % (NP - N), (NW, 1))
  pad_src = jnp.tile(jnp.arange(ppt, dtype=_i32), (NW, 1))
  srcp = jnp.concatenate([src.reshape(NW, E // NW), pad_src], axis=1).reshape(-1)
  dstp = jnp.concatenate([dst.reshape(NW, E // NW), pad_dst], axis=1).reshape(-1)
  b_c2 = b_c.reshape(1, D).astype(_f32)
  b_s2 = b_sum.reshape(1, D).astype(_f32)

  x, degp = _sc_setup(emb, tokp, dstp)
  h0p, w1 = _tc_h0(x, W_c, b_c2, degp, b_s2)
  h = h0p
  rs = []
  for _ in range(L - 1):
    sab = _sc_segsum(h, srcp, dstp)
    h, r = _tc_update(sab, h0p, w1, W_sum)
    rs.append(r)
  sab = _sc_segsum(h, srcp, dstp)
  nl, m = _tc_pack(sab, h0p, w1, W_sum, rs[0], rs[1])
  return nl, m
